# baseline (speedup 1.0000x reference)
import jax
import jax.numpy as jnp
from jax.experimental import pallas as pl
from jax.experimental.pallas import tpu as pltpu

_IN_DIM = 512
_OUT_DIM = 512
_HEADS = 2
_NEG_SLOPE = 0.2
_GROUP = 8


def _leaky(x):
    return jnp.where(x >= 0, x, _NEG_SLOPE * x)


_W_CHUNK = 256


def _gat_kernel(obj_ref, words_ref, w_hbm_ref, asrc_ref, adst_ref, bias_ref,
                out_ref, w_vmem, xlo_s, xlw_s, sems):
    S = obj_ref.shape[0]
    n_obj = obj_ref.shape[1]
    n_words = words_ref.shape[1]
    bias = bias_ref[...]
    f32 = jnp.float32
    n_chunks = (_HEADS * _OUT_DIM) // _W_CHUNK
    pid = pl.program_id(0)

    def dgT(a, b):
        return jax.lax.dot_general(a, b, (((1,), (1,)), ((), ())),
                                   preferred_element_type=f32)

    @pl.when(pid == 0)
    def _():
        for c in range(n_chunks):
            cs = pl.ds(c * _W_CHUNK, _W_CHUNK)
            pltpu.make_async_copy(w_hbm_ref.at[:, cs], w_vmem.at[:, cs],
                                  sems.at[c]).start()

    obj = obj_ref[...].reshape(S * n_obj, _IN_DIM)
    words = words_ref[...].reshape(S * n_words, _IN_DIM)
    for c in range(n_chunks):
        cs = pl.ds(c * _W_CHUNK, _W_CHUNK)

        @pl.when(pid == 0)
        def _():
            pltpu.make_async_copy(w_hbm_ref.at[:, cs], w_vmem.at[:, cs],
                                  sems.at[c]).wait()

        wc = w_vmem[:, cs]
        xlo_s[:, cs] = jnp.dot(obj, wc, preferred_element_type=f32)
        xlw_s[:, cs] = jnp.dot(words, wc, preferred_element_type=f32)
    xl_o = xlo_s[...]
    xl_w = xlw_s[...]

    heads = range(_HEADS)
    samples = range(S)
    hsl = [slice(h * _OUT_DIM, (h + 1) * _OUT_DIM) for h in heads]

    sw = {h: dgT(xl_w[:, hsl[h]], asrc_ref[h:h + 1, :]) for h in heads}
    dw = {h: dgT(xl_w[:, hsl[h]], adst_ref[h:h + 1, :]) for h in heads}
    so = {(s, h): dgT(asrc_ref[h:h + 1, :],
                      xl_o[s * n_obj:(s + 1) * n_obj, hsl[h]])
          for s in samples for h in heads}

    ex, exs = {}, {}
    for s in samples:
        for h in heads:
            adw = dw[h][s * n_words:(s + 1) * n_words]
            asw = sw[h][s * n_words:(s + 1) * n_words]
            ex[s, h] = jnp.exp(_leaky(adw + so[s, h]))
            exs[s, h] = jnp.exp(_leaky(adw + asw))

    uw, uo = {}, {}
    inv_h = 1.0 / _HEADS
    for s in samples:
        acc = None
        for h in heads:
            xo = xl_o[s * n_obj:(s + 1) * n_obj, hsl[h]]
            xw = xl_w[s * n_words:(s + 1) * n_words, hsl[h]]
            r = 1.0 / (jnp.sum(ex[s, h], axis=1, keepdims=True)
                       + exs[s, h] + 1e-16)
            num = jnp.dot(ex[s, h], xo, preferred_element_type=f32) \
                + exs[s, h] * xw
            uw_h = num * r
            acc = uw_h if acc is None else acc + uw_h
        uw[s] = acc * inv_h + bias
        ob = s * n_obj
        uo[s] = (xl_o[ob:ob + n_obj, hsl[0]]
                 + xl_o[ob:ob + n_obj, hsl[1]]) * inv_h + bias

    logits = {s: dgT(uo[s], uw[s]) for s in samples}

    v = {}
    for s in samples:
        lm = jnp.max(logits[s], axis=1, keepdims=True)
        e = jnp.exp(logits[s] - lm)
        r = 1.0 / jnp.sum(e, axis=1, keepdims=True)
        attn = e * r
        v[s] = jnp.sum(attn, axis=0, keepdims=True) * (1.0 / n_obj)

    for s in samples:
        out_ref[s] = jnp.dot(v[s], uw[s], preferred_element_type=f32)


def kernel(object_embs, word_embs, W, att_src, att_dst, bias):
    B, n_obj, in_dim = object_embs.shape
    n_words = word_embs.shape[1]
    bias2d = bias.reshape(1, _OUT_DIM)
    S = _GROUP

    return pl.pallas_call(
        _gat_kernel,
        grid=(B // S,),
        in_specs=[
            pl.BlockSpec((S, n_obj, in_dim), lambda b: (b, 0, 0)),
            pl.BlockSpec((S, n_words, in_dim), lambda b: (b, 0, 0)),
            pl.BlockSpec(memory_space=pl.ANY),
            pl.BlockSpec((_HEADS, _OUT_DIM), lambda b: (0, 0)),
            pl.BlockSpec((_HEADS, _OUT_DIM), lambda b: (0, 0)),
            pl.BlockSpec((1, _OUT_DIM), lambda b: (0, 0)),
        ],
        out_specs=pl.BlockSpec((S, 1, _OUT_DIM), lambda b: (b, 0, 0)),
        out_shape=jax.ShapeDtypeStruct((B, 1, _OUT_DIM), jnp.float32),
        scratch_shapes=[
            pltpu.VMEM((in_dim, _HEADS * _OUT_DIM), jnp.float32),
            pltpu.VMEM((S * n_obj, _HEADS * _OUT_DIM), jnp.float32),
            pltpu.VMEM((S * n_words, _HEADS * _OUT_DIM), jnp.float32),
            pltpu.SemaphoreType.DMA(((_HEADS * _OUT_DIM) // _W_CHUNK,)),
        ],
        compiler_params=pltpu.CompilerParams(
            dimension_semantics=("arbitrary",)),
    )(object_embs, word_embs, W, att_src, att_dst, bias2d).reshape(B, _OUT_DIM)


# speedup vs baseline: 1.1899x; 1.1899x over previous
import jax
import jax.numpy as jnp
from jax.experimental import pallas as pl
from jax.experimental.pallas import tpu as pltpu

_IN_DIM = 512
_OUT_DIM = 512
_HEADS = 2
_NEG_SLOPE = 0.2
_GROUP = 8


def _leaky(x):
    return jnp.where(x >= 0, x, _NEG_SLOPE * x)


def _gat_kernel(obj_ref, words_ref, w_hbm_ref, asrc_ref, adst_ref, bias_ref,
                out_ref, w_vmem, sems):
    S = obj_ref.shape[0]
    n_obj = obj_ref.shape[1]
    n_words = words_ref.shape[1]
    bias = bias_ref[...]
    f32 = jnp.float32
    pid = pl.program_id(0)
    heads = range(_HEADS)
    samples = range(S)
    hsl = [slice(h * _OUT_DIM, (h + 1) * _OUT_DIM) for h in heads]

    def dgT(a, b):
        return jax.lax.dot_general(a, b, (((1,), (1,)), ((), ())),
                                   preferred_element_type=f32)

    @pl.when(pid == 0)
    def _():
        for h in heads:
            pltpu.make_async_copy(w_hbm_ref.at[:, hsl[h]],
                                  w_vmem.at[:, hsl[h]], sems.at[h]).start()

    obj = obj_ref[...].reshape(S * n_obj, _IN_DIM)
    words = words_ref[...].reshape(S * n_words, _IN_DIM)
    xlo, xlw = {}, {}
    for h in heads:
        @pl.when(pid == 0)
        def _():
            pltpu.make_async_copy(w_hbm_ref.at[:, hsl[h]],
                                  w_vmem.at[:, hsl[h]], sems.at[h]).wait()
        wc = w_vmem[:, hsl[h]]
        xlo[h] = jnp.dot(obj, wc, preferred_element_type=f32)
        xlw[h] = jnp.dot(words, wc, preferred_element_type=f32)

    sw = {h: dgT(xlw[h], asrc_ref[h:h + 1, :]) for h in heads}
    dw = {h: dgT(xlw[h], adst_ref[h:h + 1, :]) for h in heads}
    so = {(s, h): dgT(asrc_ref[h:h + 1, :],
                      xlo[h][s * n_obj:(s + 1) * n_obj])
          for s in samples for h in heads}

    ex, exs = {}, {}
    for s in samples:
        for h in heads:
            adw = dw[h][s * n_words:(s + 1) * n_words]
            asw = sw[h][s * n_words:(s + 1) * n_words]
            ex[s, h] = jnp.exp(_leaky(adw + so[s, h]))
            exs[s, h] = jnp.exp(_leaky(adw + asw))

    uw, uo = {}, {}
    inv_h = 1.0 / _HEADS
    for s in samples:
        acc = None
        for h in heads:
            xo = xlo[h][s * n_obj:(s + 1) * n_obj]
            xw = xlw[h][s * n_words:(s + 1) * n_words]
            r = 1.0 / (jnp.sum(ex[s, h], axis=1, keepdims=True)
                       + exs[s, h] + 1e-16)
            num = jnp.dot(ex[s, h], xo, preferred_element_type=f32) \
                + exs[s, h] * xw
            uw_h = num * r
            acc = uw_h if acc is None else acc + uw_h
        uw[s] = acc * inv_h + bias
        ob = s * n_obj
        uo[s] = (xlo[0][ob:ob + n_obj]
                 + xlo[1][ob:ob + n_obj]) * inv_h + bias

    logits = {s: dgT(uo[s], uw[s]) for s in samples}

    v = {}
    for s in samples:
        lm = jnp.max(logits[s], axis=1, keepdims=True)
        e = jnp.exp(logits[s] - lm)
        r = 1.0 / jnp.sum(e, axis=1, keepdims=True)
        attn = e * r
        v[s] = jnp.sum(attn, axis=0, keepdims=True) * (1.0 / n_obj)

    for s in samples:
        out_ref[s] = jnp.dot(v[s], uw[s], preferred_element_type=f32)


def kernel(object_embs, word_embs, W, att_src, att_dst, bias):
    B, n_obj, in_dim = object_embs.shape
    n_words = word_embs.shape[1]
    bias2d = bias.reshape(1, _OUT_DIM)
    S = _GROUP

    return pl.pallas_call(
        _gat_kernel,
        grid=(B // S,),
        in_specs=[
            pl.BlockSpec((S, n_obj, in_dim), lambda b: (b, 0, 0)),
            pl.BlockSpec((S, n_words, in_dim), lambda b: (b, 0, 0)),
            pl.BlockSpec(memory_space=pl.ANY),
            pl.BlockSpec((_HEADS, _OUT_DIM), lambda b: (0, 0)),
            pl.BlockSpec((_HEADS, _OUT_DIM), lambda b: (0, 0)),
            pl.BlockSpec((1, _OUT_DIM), lambda b: (0, 0)),
        ],
        out_specs=pl.BlockSpec((S, 1, _OUT_DIM), lambda b: (b, 0, 0)),
        out_shape=jax.ShapeDtypeStruct((B, 1, _OUT_DIM), jnp.float32),
        scratch_shapes=[
            pltpu.VMEM((in_dim, _HEADS * _OUT_DIM), jnp.float32),
            pltpu.SemaphoreType.DMA((_HEADS,)),
        ],
        compiler_params=pltpu.CompilerParams(
            dimension_semantics=("arbitrary",)),
    )(object_embs, word_embs, W, att_src, att_dst, bias2d).reshape(B, _OUT_DIM)


# speedup vs baseline: 1.3448x; 1.1301x over previous
import jax
import jax.numpy as jnp
from jax.experimental import pallas as pl
from jax.experimental.pallas import tpu as pltpu

_IN_DIM = 512
_OUT_DIM = 512
_HEADS = 2
_NEG_SLOPE = 0.2
_GROUP = 8


def _leaky(x):
    return jnp.where(x >= 0, x, _NEG_SLOPE * x)


def _gat_kernel(obj_ref, words_ref, w_ref, asrc_ref, adst_ref, bias_ref,
                out_ref):
    S = obj_ref.shape[0]
    n_obj = obj_ref.shape[1]
    n_words = words_ref.shape[1]
    Wm = w_ref[...]
    bias = bias_ref[...]
    f32 = jnp.float32

    def dgT(a, b):
        return jax.lax.dot_general(a, b, (((1,), (1,)), ((), ())),
                                   preferred_element_type=f32)

    obj = obj_ref[...].reshape(S * n_obj, _IN_DIM)
    words = words_ref[...].reshape(S * n_words, _IN_DIM)
    xl_o = jnp.dot(obj, Wm, preferred_element_type=f32)
    xl_w = jnp.dot(words, Wm, preferred_element_type=f32)

    heads = range(_HEADS)
    samples = range(S)
    hsl = [slice(h * _OUT_DIM, (h + 1) * _OUT_DIM) for h in heads]

    sw = {h: dgT(xl_w[:, hsl[h]], asrc_ref[h:h + 1, :]) for h in heads}
    dw = {h: dgT(xl_w[:, hsl[h]], adst_ref[h:h + 1, :]) for h in heads}
    so = {(s, h): dgT(asrc_ref[h:h + 1, :],
                      xl_o[s * n_obj:(s + 1) * n_obj, hsl[h]])
          for s in samples for h in heads}

    ex, exs = {}, {}
    for s in samples:
        for h in heads:
            adw = dw[h][s * n_words:(s + 1) * n_words]
            asw = sw[h][s * n_words:(s + 1) * n_words]
            ex[s, h] = jnp.exp(_leaky(adw + so[s, h]))
            exs[s, h] = jnp.exp(_leaky(adw + asw))

    uw, uo = {}, {}
    inv_h = 1.0 / _HEADS
    for s in samples:
        acc = None
        for h in heads:
            xo = xl_o[s * n_obj:(s + 1) * n_obj, hsl[h]]
            xw = xl_w[s * n_words:(s + 1) * n_words, hsl[h]]
            r = inv_h / (jnp.sum(ex[s, h], axis=1, keepdims=True)
                         + exs[s, h] + 1e-16)
            num = jnp.dot(ex[s, h], xo, preferred_element_type=f32) \
                + exs[s, h] * xw
            uw_h = num * r
            acc = uw_h if acc is None else acc + uw_h
        uw[s] = acc + bias
        ob = s * n_obj
        uo[s] = (xl_o[ob:ob + n_obj, hsl[0]]
                 + xl_o[ob:ob + n_obj, hsl[1]]) * inv_h + bias

    logits = {s: dgT(uo[s], uw[s]) for s in samples}

    v = {}
    for s in samples:
        lm = jnp.max(logits[s], axis=1, keepdims=True)
        e = jnp.exp(logits[s] - lm)
        r = 1.0 / jnp.sum(e, axis=1, keepdims=True)
        attn = e * r
        v[s] = jnp.sum(attn, axis=0, keepdims=True) * (1.0 / n_obj)

    for s in samples:
        out_ref[s] = jnp.dot(v[s], uw[s], preferred_element_type=f32)


def kernel(object_embs, word_embs, W, att_src, att_dst, bias):
    B, n_obj, in_dim = object_embs.shape
    n_words = word_embs.shape[1]
    bias2d = bias.reshape(1, _OUT_DIM)
    S = _GROUP

    return pl.pallas_call(
        _gat_kernel,
        grid=(B // S,),
        in_specs=[
            pl.BlockSpec((S, n_obj, in_dim), lambda b: (b, 0, 0)),
            pl.BlockSpec((S, n_words, in_dim), lambda b: (b, 0, 0)),
            pl.BlockSpec((in_dim, _HEADS * _OUT_DIM), lambda b: (0, 0)),
            pl.BlockSpec((_HEADS, _OUT_DIM), lambda b: (0, 0)),
            pl.BlockSpec((_HEADS, _OUT_DIM), lambda b: (0, 0)),
            pl.BlockSpec((1, _OUT_DIM), lambda b: (0, 0)),
        ],
        out_specs=pl.BlockSpec((S, 1, _OUT_DIM), lambda b: (b, 0, 0)),
        out_shape=jax.ShapeDtypeStruct((B, 1, _OUT_DIM), jnp.float32),
        compiler_params=pltpu.CompilerParams(
            dimension_semantics=("parallel",)),
    )(object_embs, word_embs, W, att_src, att_dst, bias2d).reshape(B, _OUT_DIM)
